# baseline (speedup 1.0000x reference)
import functools

import jax
import jax.numpy as jnp
from jax import lax
from jax.experimental import pallas as pl
from jax.experimental.pallas import tpu as pltpu
from jax.experimental.pallas import tpu_sc as plsc

NC = 2
NS = 16
CHUNK = 128
HF = 64
K = 4
K2 = 2 * K


def _sc_segment_sum(n_pad, ch, featL, featR, src3, dst3, zagg, zdeg, ones16):
    rows_per_tile = n_pad // NS
    ni = ch // K2

    mesh = plsc.VectorSubcoreMesh(
        core_axis_name="c", subcore_axis_name="s",
        num_cores=NC, num_subcores=NS)

    @functools.partial(
        pl.kernel,
        out_type=[
            jax.ShapeDtypeStruct((NC, n_pad, HF), jnp.float32),
            jax.ShapeDtypeStruct((NC, n_pad, 16), jnp.float32),
        ],
        mesh=mesh,
        scratch_types=[
            pltpu.VMEM((2, K2, CHUNK), jnp.int32),
            pltpu.VMEM((2, K2, CHUNK), jnp.int32),
            pltpu.VMEM((K2, CHUNK, HF), jnp.float32),
            pltpu.VMEM((CHUNK, 16), jnp.float32),
            pltpu.VMEM_SHARED((n_pad, HF), jnp.float32),
            pltpu.VMEM_SHARED((n_pad, 16), jnp.float32),
            pltpu.SemaphoreType.DMA,
            pltpu.SemaphoreType.DMA,
            pltpu.SemaphoreType.DMA,
            pltpu.SemaphoreType.DMA,
            pltpu.SemaphoreType.DMA,
        ],
        compiler_params=pltpu.CompilerParams(use_tc_tiling_on_sc=False),
    )
    def sc_fn(featL_hbm, featR_hbm, src_hbm, dst_hbm,
              zagg_hbm, zdeg_hbm, o16_hbm,
              agg_out, deg_out,
              src_v, dst_v, bufs, onesbuf,
              agg_sp, deg_sp, gsemA, gsemB, ssemA, ssemB, isem):
        c = lax.axis_index("c")
        s = lax.axis_index("s")
        base = s * rows_per_tile

        pltpu.sync_copy(o16_hbm, onesbuf)
        pltpu.sync_copy(zagg_hbm, agg_sp.at[pl.ds(base, rows_per_tile)])
        pltpu.sync_copy(zdeg_hbm, deg_sp.at[pl.ds(base, rows_per_tile)])
        plsc.subcore_barrier()

        def fire_idx(m):
            sl = m % 2
            pltpu.async_copy(src_hbm.at[s, pl.ds(m * K2, K2)],
                             src_v.at[sl], isem)
            pltpu.async_copy(dst_hbm.at[s, pl.ds(m * K2, K2)],
                             dst_v.at[sl], isem)

        def wait_idx(m):
            sl = m % 2
            pltpu.make_async_copy(src_hbm.at[s, pl.ds(m * K2, K2)],
                                  src_v.at[sl], isem).wait()
            pltpu.make_async_copy(dst_hbm.at[s, pl.ds(m * K2, K2)],
                                  dst_v.at[sl], isem).wait()

        def run(feat_half, deg_par):
            def fire_gathers(par, row0):
                gsem = gsemA if row0 == 0 else gsemB
                for t in range(K):
                    pltpu.async_copy(
                        feat_half.at[src_v.at[par, row0 + t]],
                        bufs.at[row0 + t], gsem)

            def wait_gathers(par, row0):
                gsem = gsemA if row0 == 0 else gsemB
                for t in range(K):
                    pltpu.make_async_copy(
                        feat_half.at[src_v.at[par, row0 + t]],
                        bufs.at[row0 + t], gsem).wait()

            def fire_scatters(par, row0):
                sem = ssemA if row0 == 0 else ssemB
                for t in range(K):
                    pltpu.async_copy(bufs.at[row0 + t],
                                     agg_sp.at[dst_v.at[par, row0 + t]],
                                     sem, add=True)
                    if t % 2 == deg_par:
                        pltpu.async_copy(onesbuf,
                                         deg_sp.at[dst_v.at[par, row0 + t]],
                                         sem, add=True)

            def wait_scatters(par, row0):
                sem = ssemA if row0 == 0 else ssemB
                for t in range(K):
                    pltpu.make_async_copy(
                        bufs.at[row0 + t],
                        agg_sp.at[dst_v.at[par, row0 + t]], sem).wait()
                    if t % 2 == deg_par:
                        pltpu.make_async_copy(
                            onesbuf, deg_sp.at[dst_v.at[par, row0 + t]],
                            sem).wait()

            wait_idx(0)
            fire_gathers(0, 0)
            fire_gathers(0, K)

            def body(i, carry):
                par = i % 2

                @pl.when(i + 1 < ni)
                def _():
                    fire_idx(i + 1)

                for row0 in (0, K):
                    wait_gathers(par, row0)
                    fire_scatters(par, row0)
                    wait_scatters(par, row0)

                    @pl.when(i + 1 < ni)
                    def _():
                        if row0 == 0:
                            wait_idx(i + 1)
                        fire_gathers(1 - par, row0)
                return carry

            lax.fori_loop(0, ni, body, 0)

        fire_idx(0)

        @pl.when(c == 0)
        def _():
            run(featL_hbm, 0)

        @pl.when(c == 1)
        def _():
            run(featR_hbm, 1)

        plsc.subcore_barrier()

        pltpu.sync_copy(agg_sp.at[pl.ds(base, rows_per_tile)],
                        agg_out.at[c, pl.ds(base, rows_per_tile)])
        pltpu.sync_copy(deg_sp.at[pl.ds(base, rows_per_tile)],
                        deg_out.at[c, pl.ds(base, rows_per_tile)])

    return sc_fn(featL, featR, src3, dst3, zagg, zdeg, ones16)


def _tc_epilogue(n, n_pad, aggL, aggR, deg0, deg1, WL, WR, b2d):
    blk = 1024
    dn = (((1,), (1,)), ((), ()))

    def body(a0, a1, d0, d1, wl, wr, bv, o):
        deg = d0[...][:, 0:1] + d1[...][:, 0:1]
        o[...] = (
            lax.dot_general(a0[...], wl[...], dn,
                            preferred_element_type=jnp.float32)
            + lax.dot_general(a1[...], wr[...], dn,
                              preferred_element_type=jnp.float32)
            + deg * bv[...])

    return pl.pallas_call(
        body,
        grid=(n_pad // blk,),
        in_specs=[
            pl.BlockSpec((blk, HF), lambda i: (i, 0)),
            pl.BlockSpec((blk, HF), lambda i: (i, 0)),
            pl.BlockSpec((blk, 16), lambda i: (i, 0)),
            pl.BlockSpec((blk, 16), lambda i: (i, 0)),
            pl.BlockSpec((128, HF), lambda i: (0, 0)),
            pl.BlockSpec((128, HF), lambda i: (0, 0)),
            pl.BlockSpec((1, 128), lambda i: (0, 0)),
        ],
        out_specs=pl.BlockSpec((blk, 128), lambda i: (i, 0)),
        out_shape=jax.ShapeDtypeStruct((n, 128), jnp.float32),
    )(aggL, aggR, deg0, deg1, WL, WR, b2d)


def kernel(feat, edge_index, W, b):
    n = feat.shape[0]
    e = edge_index.shape[1]
    n_pad = ((n + 2047) // 2048) * 2048
    gsz = K2 * CHUNK
    epw = gsz * (-(-e // (NS * gsz)))
    e_pad = NS * epw
    ch = epw // CHUNK

    src = edge_index[0].astype(jnp.int32)
    dst = edge_index[1].astype(jnp.int32)
    src3 = jnp.concatenate(
        [src, jnp.zeros((e_pad - e,), jnp.int32)]).reshape(NS, ch, CHUNK)
    dst3 = jnp.concatenate(
        [dst, jnp.full((e_pad - e,), n, jnp.int32)]).reshape(NS, ch, CHUNK)

    featL = feat[:, :HF]
    featR = feat[:, HF:]
    zagg = jnp.zeros((n_pad // NS, HF), jnp.float32)
    zdeg = jnp.zeros((n_pad // NS, 16), jnp.float32)
    ones16 = jnp.ones((CHUNK, 16), jnp.float32)

    agg, deg = _sc_segment_sum(n_pad, ch, featL, featR, src3, dst3,
                               zagg, zdeg, ones16)
    return _tc_epilogue(n, n_pad, agg[0], agg[1], deg[0], deg[1],
                        W[:, :HF], W[:, HF:], b.reshape(1, -1))


# speedup vs baseline: 1.0042x; 1.0042x over previous
import functools

import jax
import jax.numpy as jnp
from jax import lax
from jax.experimental import pallas as pl
from jax.experimental.pallas import tpu as pltpu
from jax.experimental.pallas import tpu_sc as plsc

NC = 2
NS = 16
CHUNK = 128
HF = 64
K = 2
K2 = 2 * K


def _sc_segment_sum(n_pad, ch, featL, featR, src3, dst3, zagg, zdeg, ones16):
    rows_per_tile = n_pad // NS
    ni = ch // K2

    mesh = plsc.VectorSubcoreMesh(
        core_axis_name="c", subcore_axis_name="s",
        num_cores=NC, num_subcores=NS)

    @functools.partial(
        pl.kernel,
        out_type=[
            jax.ShapeDtypeStruct((NC, n_pad, HF), jnp.float32),
            jax.ShapeDtypeStruct((NC, n_pad, 16), jnp.float32),
        ],
        mesh=mesh,
        scratch_types=[
            pltpu.VMEM((2, K2, CHUNK), jnp.int32),
            pltpu.VMEM((2, K2, CHUNK), jnp.int32),
            pltpu.VMEM((K2, CHUNK, HF), jnp.float32),
            pltpu.VMEM((CHUNK, 16), jnp.float32),
            pltpu.VMEM_SHARED((n_pad, HF), jnp.float32),
            pltpu.VMEM_SHARED((n_pad, 16), jnp.float32),
            pltpu.SemaphoreType.DMA,
            pltpu.SemaphoreType.DMA,
            pltpu.SemaphoreType.DMA,
            pltpu.SemaphoreType.DMA,
            pltpu.SemaphoreType.DMA,
        ],
        compiler_params=pltpu.CompilerParams(use_tc_tiling_on_sc=False),
    )
    def sc_fn(featL_hbm, featR_hbm, src_hbm, dst_hbm,
              zagg_hbm, zdeg_hbm, o16_hbm,
              agg_out, deg_out,
              src_v, dst_v, bufs, onesbuf,
              agg_sp, deg_sp, gsemA, gsemB, ssemA, ssemB, isem):
        c = lax.axis_index("c")
        s = lax.axis_index("s")
        base = s * rows_per_tile

        pltpu.sync_copy(o16_hbm, onesbuf)
        pltpu.sync_copy(zagg_hbm, agg_sp.at[pl.ds(base, rows_per_tile)])
        pltpu.sync_copy(zdeg_hbm, deg_sp.at[pl.ds(base, rows_per_tile)])
        plsc.subcore_barrier()

        def fire_idx(m):
            sl = m % 2
            pltpu.async_copy(src_hbm.at[s, pl.ds(m * K2, K2)],
                             src_v.at[sl], isem)
            pltpu.async_copy(dst_hbm.at[s, pl.ds(m * K2, K2)],
                             dst_v.at[sl], isem)

        def wait_idx(m):
            sl = m % 2
            pltpu.make_async_copy(src_hbm.at[s, pl.ds(m * K2, K2)],
                                  src_v.at[sl], isem).wait()
            pltpu.make_async_copy(dst_hbm.at[s, pl.ds(m * K2, K2)],
                                  dst_v.at[sl], isem).wait()

        def run(feat_half, deg_par):
            def fire_gathers(par, row0):
                gsem = gsemA if row0 == 0 else gsemB
                for t in range(K):
                    pltpu.async_copy(
                        feat_half.at[src_v.at[par, row0 + t]],
                        bufs.at[row0 + t], gsem)

            def wait_gathers(par, row0):
                gsem = gsemA if row0 == 0 else gsemB
                for t in range(K):
                    pltpu.make_async_copy(
                        feat_half.at[src_v.at[par, row0 + t]],
                        bufs.at[row0 + t], gsem).wait()

            def fire_scatters(par, row0):
                sem = ssemA if row0 == 0 else ssemB
                for t in range(K):
                    pltpu.async_copy(bufs.at[row0 + t],
                                     agg_sp.at[dst_v.at[par, row0 + t]],
                                     sem, add=True)
                    if (row0 + t) % 2 == deg_par:
                        pltpu.async_copy(onesbuf,
                                         deg_sp.at[dst_v.at[par, row0 + t]],
                                         sem, add=True)

            def wait_scatters(par, row0):
                sem = ssemA if row0 == 0 else ssemB
                for t in range(K):
                    pltpu.make_async_copy(
                        bufs.at[row0 + t],
                        agg_sp.at[dst_v.at[par, row0 + t]], sem).wait()
                    if (row0 + t) % 2 == deg_par:
                        pltpu.make_async_copy(
                            onesbuf, deg_sp.at[dst_v.at[par, row0 + t]],
                            sem).wait()

            wait_idx(0)
            fire_gathers(0, 0)
            fire_gathers(0, K)

            def body(i, carry):
                par = i % 2

                @pl.when(i + 1 < ni)
                def _():
                    fire_idx(i + 1)

                for row0 in (0, K):
                    wait_gathers(par, row0)
                    fire_scatters(par, row0)
                    wait_scatters(par, row0)

                    @pl.when(i + 1 < ni)
                    def _():
                        if row0 == 0:
                            wait_idx(i + 1)
                        fire_gathers(1 - par, row0)
                return carry

            lax.fori_loop(0, ni, body, 0)

        fire_idx(0)

        @pl.when(c == 0)
        def _():
            run(featL_hbm, 0)

        @pl.when(c == 1)
        def _():
            run(featR_hbm, 1)

        plsc.subcore_barrier()

        pltpu.sync_copy(agg_sp.at[pl.ds(base, rows_per_tile)],
                        agg_out.at[c, pl.ds(base, rows_per_tile)])
        pltpu.sync_copy(deg_sp.at[pl.ds(base, rows_per_tile)],
                        deg_out.at[c, pl.ds(base, rows_per_tile)])

    return sc_fn(featL, featR, src3, dst3, zagg, zdeg, ones16)


def _tc_epilogue(n, n_pad, aggL, aggR, deg0, deg1, WL, WR, b2d):
    blk = 1024
    dn = (((1,), (1,)), ((), ()))

    def body(a0, a1, d0, d1, wl, wr, bv, o):
        deg = d0[...][:, 0:1] + d1[...][:, 0:1]
        o[...] = (
            lax.dot_general(a0[...], wl[...], dn,
                            preferred_element_type=jnp.float32)
            + lax.dot_general(a1[...], wr[...], dn,
                              preferred_element_type=jnp.float32)
            + deg * bv[...])

    return pl.pallas_call(
        body,
        grid=(n_pad // blk,),
        in_specs=[
            pl.BlockSpec((blk, HF), lambda i: (i, 0)),
            pl.BlockSpec((blk, HF), lambda i: (i, 0)),
            pl.BlockSpec((blk, 16), lambda i: (i, 0)),
            pl.BlockSpec((blk, 16), lambda i: (i, 0)),
            pl.BlockSpec((128, HF), lambda i: (0, 0)),
            pl.BlockSpec((128, HF), lambda i: (0, 0)),
            pl.BlockSpec((1, 128), lambda i: (0, 0)),
        ],
        out_specs=pl.BlockSpec((blk, 128), lambda i: (i, 0)),
        out_shape=jax.ShapeDtypeStruct((n, 128), jnp.float32),
    )(aggL, aggR, deg0, deg1, WL, WR, b2d)


def kernel(feat, edge_index, W, b):
    n = feat.shape[0]
    e = edge_index.shape[1]
    n_pad = ((n + 2047) // 2048) * 2048
    gsz = K2 * CHUNK
    epw = gsz * (-(-e // (NS * gsz)))
    e_pad = NS * epw
    ch = epw // CHUNK

    src = edge_index[0].astype(jnp.int32)
    dst = edge_index[1].astype(jnp.int32)
    src3 = jnp.concatenate(
        [src, jnp.zeros((e_pad - e,), jnp.int32)]).reshape(NS, ch, CHUNK)
    dst3 = jnp.concatenate(
        [dst, jnp.full((e_pad - e,), n, jnp.int32)]).reshape(NS, ch, CHUNK)

    featL = feat[:, :HF]
    featR = feat[:, HF:]
    zagg = jnp.zeros((n_pad // NS, HF), jnp.float32)
    zdeg = jnp.zeros((n_pad // NS, 16), jnp.float32)
    ones16 = jnp.ones((CHUNK, 16), jnp.float32)

    agg, deg = _sc_segment_sum(n_pad, ch, featL, featR, src3, dst3,
                               zagg, zdeg, ones16)
    return _tc_epilogue(n, n_pad, agg[0], agg[1], deg[0], deg[1],
                        W[:, :HF], W[:, HF:], b.reshape(1, -1))


# speedup vs baseline: 1.2763x; 1.2709x over previous
import functools

import jax
import jax.numpy as jnp
from jax import lax
from jax.experimental import pallas as pl
from jax.experimental.pallas import tpu as pltpu
from jax.experimental.pallas import tpu_sc as plsc

NC = 2
NS = 16
CHUNK = 128
HF = 64
K = 2
K2 = 2 * K


def _sc_segment_sum(n_pad, ch, featL, featR, src3, dst3, zagg, zdeg, ones16):
    rows_per_tile = n_pad // NS
    ni = ch // K2

    mesh = plsc.VectorSubcoreMesh(
        core_axis_name="c", subcore_axis_name="s",
        num_cores=NC, num_subcores=NS)

    @functools.partial(
        pl.kernel,
        out_type=[
            jax.ShapeDtypeStruct((NC, n_pad, HF), jnp.float32),
            jax.ShapeDtypeStruct((NC, n_pad, 16), jnp.float32),
        ],
        mesh=mesh,
        scratch_types=[
            pltpu.VMEM((ch, CHUNK), jnp.int32),
            pltpu.VMEM((ch, CHUNK), jnp.int32),
            pltpu.VMEM((CHUNK, HF), jnp.float32),
            pltpu.VMEM((CHUNK, HF), jnp.float32),
            pltpu.VMEM((CHUNK, 16), jnp.float32),
            pltpu.VMEM_SHARED((n_pad, HF), jnp.float32),
            pltpu.VMEM_SHARED((n_pad, 16), jnp.float32),
            pltpu.SemaphoreType.DMA,
            pltpu.SemaphoreType.DMA,
            pltpu.SemaphoreType.DMA,
        ],
        compiler_params=pltpu.CompilerParams(use_tc_tiling_on_sc=False),
    )
    def sc_fn(featL_hbm, featR_hbm, src_hbm, dst_hbm,
              zagg_hbm, zdeg_hbm, o16_hbm,
              agg_out, deg_out,
              src_v, dst_v, rowbuf0, rowbuf1, onesbuf,
              agg_sp, deg_sp, gsem0, gsem1, ssem):
        c = lax.axis_index("c")
        s = lax.axis_index("s")
        base = s * rows_per_tile

        pltpu.sync_copy(o16_hbm, onesbuf)
        pltpu.sync_copy(src_hbm.at[s], src_v)
        pltpu.sync_copy(dst_hbm.at[s], dst_v)
        pltpu.sync_copy(zagg_hbm, agg_sp.at[pl.ds(base, rows_per_tile)])
        pltpu.sync_copy(zdeg_hbm, deg_sp.at[pl.ds(base, rows_per_tile)])
        plsc.subcore_barrier()

        def run(feat_half, deg_par):
            pltpu.async_copy(feat_half.at[src_v.at[0]], rowbuf0, gsem0)
            pltpu.async_copy(feat_half.at[src_v.at[1]], rowbuf1, gsem1)

            def body(g, carry):
                j0 = g * 2
                for par, buf, gsem in ((0, rowbuf0, gsem0), (1, rowbuf1, gsem1)):
                    j = j0 + par
                    pltpu.make_async_copy(feat_half.at[src_v.at[j]],
                                          buf, gsem).wait()
                    pltpu.async_copy(buf, agg_sp.at[dst_v.at[j]], ssem,
                                     add=True)
                    if deg_par == par:
                        pltpu.async_copy(onesbuf, deg_sp.at[dst_v.at[j]],
                                         ssem, add=True)
                    pltpu.make_async_copy(buf, agg_sp.at[dst_v.at[j]],
                                          ssem).wait()
                    if deg_par == par:
                        pltpu.make_async_copy(onesbuf, deg_sp.at[dst_v.at[j]],
                                              ssem).wait()

                    @pl.when(j + 2 < ch)
                    def _():
                        pltpu.async_copy(feat_half.at[src_v.at[j + 2]],
                                         buf, gsem)
                return carry

            lax.fori_loop(0, ch // 2, body, 0)

        @pl.when(c == 0)
        def _():
            run(featL_hbm, 0)

        @pl.when(c == 1)
        def _():
            run(featR_hbm, 1)

        plsc.subcore_barrier()

        pltpu.sync_copy(agg_sp.at[pl.ds(base, rows_per_tile)],
                        agg_out.at[c, pl.ds(base, rows_per_tile)])
        pltpu.sync_copy(deg_sp.at[pl.ds(base, rows_per_tile)],
                        deg_out.at[c, pl.ds(base, rows_per_tile)])

    return sc_fn(featL, featR, src3, dst3, zagg, zdeg, ones16)


def _tc_epilogue(n, n_pad, aggL, aggR, deg0, deg1, WL, WR, b2d):
    blk = 1024
    dn = (((1,), (1,)), ((), ()))

    def body(a0, a1, d0, d1, wl, wr, bv, o):
        deg = d0[...][:, 0:1] + d1[...][:, 0:1]
        o[...] = (
            lax.dot_general(a0[...], wl[...], dn,
                            preferred_element_type=jnp.float32)
            + lax.dot_general(a1[...], wr[...], dn,
                              preferred_element_type=jnp.float32)
            + deg * bv[...])

    return pl.pallas_call(
        body,
        grid=(n_pad // blk,),
        in_specs=[
            pl.BlockSpec((blk, HF), lambda i: (i, 0)),
            pl.BlockSpec((blk, HF), lambda i: (i, 0)),
            pl.BlockSpec((blk, 16), lambda i: (i, 0)),
            pl.BlockSpec((blk, 16), lambda i: (i, 0)),
            pl.BlockSpec((128, HF), lambda i: (0, 0)),
            pl.BlockSpec((128, HF), lambda i: (0, 0)),
            pl.BlockSpec((1, 128), lambda i: (0, 0)),
        ],
        out_specs=pl.BlockSpec((blk, 128), lambda i: (i, 0)),
        out_shape=jax.ShapeDtypeStruct((n, 128), jnp.float32),
    )(aggL, aggR, deg0, deg1, WL, WR, b2d)


def kernel(feat, edge_index, W, b):
    n = feat.shape[0]
    e = edge_index.shape[1]
    n_pad = ((n + 2047) // 2048) * 2048
    gsz = 2 * CHUNK
    epw = gsz * (-(-e // (NS * gsz)))
    e_pad = NS * epw
    ch = epw // CHUNK

    src = edge_index[0].astype(jnp.int32)
    dst = edge_index[1].astype(jnp.int32)
    src3 = jnp.concatenate(
        [src, jnp.zeros((e_pad - e,), jnp.int32)]).reshape(NS, ch, CHUNK)
    dst3 = jnp.concatenate(
        [dst, jnp.full((e_pad - e,), n, jnp.int32)]).reshape(NS, ch, CHUNK)

    featL = feat[:, :HF]
    featR = feat[:, HF:]
    zagg = jnp.zeros((n_pad // NS, HF), jnp.float32)
    zdeg = jnp.zeros((n_pad // NS, 16), jnp.float32)
    ones16 = jnp.ones((CHUNK, 16), jnp.float32)

    agg, deg = _sc_segment_sum(n_pad, ch, featL, featR, src3, dst3,
                               zagg, zdeg, ones16)
    return _tc_epilogue(n, n_pad, agg[0], agg[1], deg[0], deg[1],
                        W[:, :HF], W[:, HF:], b.reshape(1, -1))
